# Initial kernel scaffold; baseline (speedup 1.0000x reference)
import jax
import jax.numpy as jnp
from jax.experimental import pallas as pl


def kernel(X1, X2, row_map, Wq, bq, Wk, bk, Wv, bv):
    raise NotImplementedError("write your pallas kernel here")


# speedup vs baseline: 41.7229x; 41.7229x over previous
import functools

import jax
import jax.numpy as jnp
import numpy as np
from jax import lax
from jax.experimental import pallas as pl
from jax.experimental.pallas import tpu as pltpu
from jax.experimental.pallas import tpu_sc as plsc

_NQ = 4096
_NK = 16384
_QDI = 64
_QDO = 32
_VDI = 64
_VDO = 32
_KVD = _QDO + _VDO

_NC = 2
_NS = 16
_NW = _NC * _NS
_KPW = _NK // _NW
_CH = 128
_NCH = _KPW // _CH
_CW = _VDO + 16
_RPT = _NQ // _NS


def _sc_mesh():
    return plsc.VectorSubcoreMesh(
        core_axis_name="c", subcore_axis_name="s",
        num_cores=_NC, num_subcores=_NS)


_SC_PARAMS = pltpu.CompilerParams(use_tc_tiling_on_sc=False)


def _proj_body(x1_ref, x2_ref, wq_ref, bq_ref, wkv_ref, bkv_ref, q_ref, kv_ref):
    q_ref[...] = (
        jnp.dot(x1_ref[...], wq_ref[...], preferred_element_type=jnp.float32)
        + bq_ref[...])
    kv_ref[...] = (
        jnp.dot(x2_ref[...], wkv_ref[...], preferred_element_type=jnp.float32)
        + bkv_ref[...])


def _tc_proj(X1, X2, Wq, bq2, Wkv, bkv2):
    g = 4
    return pl.pallas_call(
        _proj_body,
        grid=(g,),
        in_specs=[
            pl.BlockSpec((_NQ // g, _QDI), lambda i: (i, 0)),
            pl.BlockSpec((_NK // g, _VDI), lambda i: (i, 0)),
            pl.BlockSpec((_QDI, _QDO), lambda i: (0, 0)),
            pl.BlockSpec((1, _QDO), lambda i: (0, 0)),
            pl.BlockSpec((_VDI, _KVD), lambda i: (0, 0)),
            pl.BlockSpec((1, _KVD), lambda i: (0, 0)),
        ],
        out_specs=[
            pl.BlockSpec((_NQ // g, _QDO), lambda i: (i, 0)),
            pl.BlockSpec((_NK // g, _KVD), lambda i: (i, 0)),
        ],
        out_shape=[
            jax.ShapeDtypeStruct((_NQ, _QDO), jnp.float32),
            jax.ShapeDtypeStruct((_NK, _KVD), jnp.float32),
        ],
    )(X1, X2, Wq, bq2, Wkv, bkv2)


def _sc_gather(Q, rm3):
    @functools.partial(
        pl.kernel,
        out_type=jax.ShapeDtypeStruct((_NW, _KPW, _QDO), jnp.float32),
        mesh=_sc_mesh(),
        scratch_types=[
            pltpu.VMEM((_NCH, _CH), jnp.int32),
            pltpu.VMEM((_KPW, _QDO), jnp.float32),
            pltpu.SemaphoreType.DMA,
        ],
        compiler_params=_SC_PARAMS,
    )
    def k(q_hbm, rm_hbm, out_hbm, idx_v, rows_v, sem):
        c = lax.axis_index("c")
        s = lax.axis_index("s")
        w = s * _NC + c
        pltpu.sync_copy(rm_hbm.at[w], idx_v)
        for j in range(_NCH):
            pltpu.async_copy(
                q_hbm.at[idx_v.at[j]], rows_v.at[pl.ds(j * _CH, _CH)], sem
            ).wait()
        pltpu.sync_copy(rows_v, out_hbm.at[w])

    return k(Q, rm3)


def _score_body(qg_ref, kv_ref, ct_ref):
    qg = qg_ref[...]
    kv = kv_ref[...]
    kk = kv[:, :_QDO]
    vv = kv[:, _QDO:]
    sv = jnp.sum(qg * kk, axis=1, keepdims=True) * np.float32(
        1.0 / np.sqrt(_QDO))
    m = jnp.max(sv)
    e = jnp.exp(sv - m)
    ev = e * vv
    eb = jnp.broadcast_to(e, (_NK, _CW - _VDO))
    ct_ref[...] = jnp.concatenate([ev, eb], axis=1)


def _tc_score(Qg, KV):
    return pl.pallas_call(
        _score_body,
        in_specs=[
            pl.BlockSpec((_NK, _QDO), lambda: (0, 0)),
            pl.BlockSpec((_NK, _KVD), lambda: (0, 0)),
        ],
        out_specs=pl.BlockSpec((_NK, _CW), lambda: (0, 0)),
        out_shape=jax.ShapeDtypeStruct((_NK, _CW), jnp.float32),
    )(Qg, KV)


def _sc_scatter(ct3, rm3, zeros_hbm):
    @functools.partial(
        pl.kernel,
        out_type=jax.ShapeDtypeStruct((_NW, _RPT, _CW), jnp.float32),
        mesh=_sc_mesh(),
        scratch_types=[
            pltpu.VMEM((_NCH, _CH), jnp.int32),
            pltpu.VMEM((_KPW, _CW), jnp.float32),
            pltpu.VMEM_SHARED((_NQ, _CW), jnp.float32),
        ],
        compiler_params=_SC_PARAMS,
    )
    def k(ct_hbm, rm_hbm, z_hbm, out_hbm, idx_v, ct_v, acc_sh):
        c = lax.axis_index("c")
        s = lax.axis_index("s")
        w = s * _NC + c
        pltpu.sync_copy(z_hbm.at[pl.ds(s * _RPT, _RPT)],
                        acc_sh.at[pl.ds(s * _RPT, _RPT)])
        pltpu.sync_copy(rm_hbm.at[w], idx_v)
        pltpu.sync_copy(ct_hbm.at[w], ct_v)
        plsc.subcore_barrier()
        for j in range(_NCH):
            pltpu.sync_copy(ct_v.at[pl.ds(j * _CH, _CH)],
                            acc_sh.at[idx_v.at[j]], add=True)
        plsc.subcore_barrier()
        pltpu.sync_copy(acc_sh.at[pl.ds(s * _RPT, _RPT)],
                        out_hbm.at[c * _NS + s])

    return k(ct3, rm3, zeros_hbm)


def _combine_body(p0_ref, p1_ref, c_ref):
    p = p0_ref[...] + p1_ref[...]
    num = p[:, :_VDO]
    den = p[:, _VDO:_VDO + 1]
    c_ref[...] = jnp.where(den != 0.0, num / den, 0.0)


def _tc_combine(P0, P1):
    return pl.pallas_call(
        _combine_body,
        in_specs=[
            pl.BlockSpec((_NQ, _CW), lambda: (0, 0)),
            pl.BlockSpec((_NQ, _CW), lambda: (0, 0)),
        ],
        out_specs=pl.BlockSpec((_NQ, _VDO), lambda: (0, 0)),
        out_shape=jax.ShapeDtypeStruct((_NQ, _VDO), jnp.float32),
    )(P0, P1)


def kernel(X1, X2, row_map, Wq, bq, Wk, bk, Wv, bv):
    Wkv = jnp.concatenate([Wk, Wv], axis=1)
    bkv2 = jnp.concatenate([bk, bv]).reshape(1, _KVD)
    bq2 = bq.reshape(1, _QDO)

    Q, KV = _tc_proj(X1, X2, Wq, bq2, Wkv, bkv2)

    rm3 = row_map.reshape(_NW, _NCH, _CH)
    Qg = _sc_gather(Q, rm3).reshape(_NK, _QDO)

    contrib = _tc_score(Qg, KV)

    ct3 = contrib.reshape(_NW, _KPW, _CW)
    zeros_hbm = jnp.zeros((_NQ, _CW), dtype=jnp.float32)
    P = _sc_scatter(ct3, rm3, zeros_hbm).reshape(_NC, _NQ, _CW)

    return _tc_combine(P[0], P[1])


# speedup vs baseline: 46.7616x; 1.1208x over previous
import functools

import jax
import jax.numpy as jnp
import numpy as np
from jax import lax
from jax.experimental import pallas as pl
from jax.experimental.pallas import tpu as pltpu
from jax.experimental.pallas import tpu_sc as plsc

_NQ = 4096
_NK = 16384
_QDI = 64
_QDO = 32
_VDI = 64
_VDO = 32
_KVD = _QDO + _VDO
_L = 16

_NC = 2
_NS = 16
_NW = _NC * _NS
_KPW = _NK // _NW
_CH = 128
_NCH = _KPW // _CH
_CW = _VDO + _L
_RPT = _NQ // _NS
_G = _KPW // _L

_ISQ = np.float32(1.0 / np.sqrt(np.float32(_QDO)))

_SC_PARAMS = pltpu.CompilerParams(use_tc_tiling_on_sc=False, needs_layout_passes=False)


def _sc_mesh():
    return plsc.VectorSubcoreMesh(
        core_axis_name="c", subcore_axis_name="s",
        num_cores=_NC, num_subcores=_NS)


def _proj_body(x1_ref, x2_ref, wq_ref, bq_ref, wkv_ref, bkv_ref, q_ref, kv_ref):
    q_ref[...] = (
        jnp.dot(x1_ref[...], wq_ref[...], preferred_element_type=jnp.float32)
        + bq_ref[...])
    kv_ref[...] = (
        jnp.dot(x2_ref[...], wkv_ref[...], preferred_element_type=jnp.float32)
        + bkv_ref[...])


def _tc_proj(X1, X2, Wq, bq2, Wkv, bkv2):
    g = 4
    return pl.pallas_call(
        _proj_body,
        grid=(g,),
        in_specs=[
            pl.BlockSpec((_NQ // g, _QDI), lambda i: (i, 0)),
            pl.BlockSpec((_NK // g, _VDI), lambda i: (i, 0)),
            pl.BlockSpec((_QDI, _QDO), lambda i: (0, 0)),
            pl.BlockSpec((1, _QDO), lambda i: (0, 0)),
            pl.BlockSpec((_VDI, _KVD), lambda i: (0, 0)),
            pl.BlockSpec((1, _KVD), lambda i: (0, 0)),
        ],
        out_specs=[
            pl.BlockSpec((_NQ // g, _QDO), lambda i: (i, 0)),
            pl.BlockSpec((_NK // g, _KVD), lambda i: (i, 0)),
        ],
        out_shape=[
            jax.ShapeDtypeStruct((_NQ, _QDO), jnp.float32),
            jax.ShapeDtypeStruct((_NK, _KVD), jnp.float32),
        ],
    )(X1, X2, Wq, bq2, Wkv, bkv2)


def _sc_attend(Q, KV3, rm3, zeros_hbm):
    @functools.partial(
        pl.kernel,
        out_type=(
            jax.ShapeDtypeStruct((_NW, _RPT, _CW), jnp.float32),
            jax.ShapeDtypeStruct((_NC, _L), jnp.float32),
        ),
        mesh=_sc_mesh(),
        scratch_types=[
            pltpu.VMEM((_NCH, _CH), jnp.int32),
            pltpu.VMEM((_KPW, _QDO), jnp.float32),
            pltpu.VMEM((_KPW, _KVD), jnp.float32),
            pltpu.VMEM((_KPW,), jnp.float32),
            pltpu.VMEM((_KPW, _CW), jnp.float32),
            pltpu.VMEM((_L,), jnp.float32),
            pltpu.VMEM((_NS, _L), jnp.float32),
            pltpu.VMEM_SHARED((_NS, _L), jnp.float32),
            pltpu.VMEM_SHARED((_NQ, _CW), jnp.float32),
            pltpu.SemaphoreType.DMA,
        ],
        compiler_params=_SC_PARAMS,
    )
    def k(q_hbm, kv_hbm, rm_hbm, z_hbm, out_hbm, mx_hbm,
          idx_v, qg_v, kv_v, s_v, ct_v, mymax_v, allmax_v, mx_sh, acc_sh, sem):
        c = lax.axis_index("c")
        s = lax.axis_index("s")
        w = s * _NC + c

        pltpu.sync_copy(z_hbm.at[pl.ds(s * _RPT, _RPT)],
                        acc_sh.at[pl.ds(s * _RPT, _RPT)])
        pltpu.sync_copy(rm_hbm.at[w], idx_v)
        pltpu.sync_copy(kv_hbm.at[w], kv_v)
        for j in range(_NCH):
            pltpu.async_copy(
                q_hbm.at[idx_v.at[j]], qg_v.at[pl.ds(j * _CH, _CH)], sem
            ).wait()

        lane = lax.iota(jnp.int32, _L)
        last = lane == (_L - 1)

        def dot_body(j, carry):
            p = (qg_v[j, pl.ds(0, _L)] * kv_v[j, pl.ds(0, _L)]
                 + qg_v[j, pl.ds(_L, _L)] * kv_v[j, pl.ds(_L, _L)])
            tot = plsc.cumsum(p) * _ISQ
            plsc.store_scatter(s_v, [jnp.full((_L,), j, jnp.int32)], tot,
                               mask=last)
            return carry

        lax.fori_loop(0, _KPW, dot_body, 0, unroll=2)

        def max_body(g, mx):
            return jnp.maximum(mx, s_v[pl.ds(g * _L, _L)])

        mx = lax.fori_loop(1, _G, max_body, s_v[pl.ds(0, _L)])
        mymax_v[...] = jnp.broadcast_to(jnp.max(mx), (_L,))

        pltpu.sync_copy(mymax_v, mx_sh.at[s])
        plsc.subcore_barrier()
        pltpu.sync_copy(mx_sh, allmax_v)

        def cmax_body(t, mx):
            return jnp.maximum(mx, allmax_v[t, :])

        mcore = lax.fori_loop(1, _NS, cmax_body, allmax_v[0, :])

        @pl.when(s == 0)
        def _():
            mymax_v[...] = mcore
            pltpu.sync_copy(mymax_v, mx_hbm.at[c])

        def ct_body(g, carry):
            e16 = jnp.exp(s_v[pl.ds(g * _L, _L)] - mcore)
            plsc.store_scatter(s_v, [lane + g * _L], e16)
            return carry

        lax.fori_loop(0, _G, ct_body, 0)

        def row_body(j, carry):
            esp = plsc.load_gather(s_v, [jnp.full((_L,), j, jnp.int32)])
            ct_v[j, pl.ds(0, _L)] = esp * kv_v[j, pl.ds(_QDO, _L)]
            ct_v[j, pl.ds(_L, _L)] = esp * kv_v[j, pl.ds(_QDO + _L, _L)]
            ct_v[j, pl.ds(2 * _L, _L)] = esp
            return carry

        lax.fori_loop(0, _KPW, row_body, 0, unroll=2)

        plsc.subcore_barrier()
        for j in range(_NCH):
            pltpu.sync_copy(ct_v.at[pl.ds(j * _CH, _CH)],
                            acc_sh.at[idx_v.at[j]], add=True)
        plsc.subcore_barrier()
        pltpu.sync_copy(acc_sh.at[pl.ds(s * _RPT, _RPT)],
                        out_hbm.at[c * _NS + s])

    return k(Q, KV3, rm3, zeros_hbm)


def _combine_body(p0_ref, p1_ref, mx_ref, c_ref):
    m0 = jnp.max(mx_ref[0:1, :])
    m1 = jnp.max(mx_ref[1:2, :])
    mg = jnp.maximum(m0, m1)
    a0 = jnp.exp(m0 - mg)
    a1 = jnp.exp(m1 - mg)
    p = a0 * p0_ref[...] + a1 * p1_ref[...]
    num = p[:, :_VDO]
    den = p[:, _VDO:_VDO + 1]
    c_ref[...] = jnp.where(den != 0.0, num / den, 0.0)


def _tc_combine(P0, P1, MX):
    return pl.pallas_call(
        _combine_body,
        in_specs=[
            pl.BlockSpec((_NQ, _CW), lambda: (0, 0)),
            pl.BlockSpec((_NQ, _CW), lambda: (0, 0)),
            pl.BlockSpec((_NC, _L), lambda: (0, 0)),
        ],
        out_specs=pl.BlockSpec((_NQ, _VDO), lambda: (0, 0)),
        out_shape=jax.ShapeDtypeStruct((_NQ, _VDO), jnp.float32),
    )(P0, P1, MX)


def kernel(X1, X2, row_map, Wq, bq, Wk, bk, Wv, bv):
    Wkv = jnp.concatenate([Wk, Wv], axis=1)
    bkv2 = jnp.concatenate([bk, bv]).reshape(1, _KVD)
    bq2 = bq.reshape(1, _QDO)

    Q, KV = _tc_proj(X1, X2, Wq, bq2, Wkv, bkv2)

    rm3 = row_map.reshape(_NW, _NCH, _CH)
    KV3 = KV.reshape(_NW, _KPW, _KVD)
    zeros_hbm = jnp.zeros((_NQ, _CW), dtype=jnp.float32)

    P, MX = _sc_attend(Q, KV3, rm3, zeros_hbm)
    P = P.reshape(_NC, _NQ, _CW)

    return _tc_combine(P[0], P[1], MX)


# speedup vs baseline: 77.5898x; 1.6593x over previous
import functools

import jax
import jax.numpy as jnp
import numpy as np
from jax import lax
from jax.experimental import pallas as pl
from jax.experimental.pallas import tpu as pltpu
from jax.experimental.pallas import tpu_sc as plsc

_NQ = 4096
_NK = 16384
_QDI = 64
_QDO = 32
_VDI = 64
_VDO = 32
_KVD = _QDO + _VDO
_L = 16
_W = 128

_NC = 2
_NS = 16
_NW = _NC * _NS
_KPW = _NK // _NW
_CH = 128
_NCH = _KPW // _CH
_CW = _VDO + _L
_RPT = _NQ // _NS
_G = _KPW // _L

_ISQ = np.float32(1.0 / np.sqrt(np.float32(_QDO)))

_SC_PARAMS = pltpu.CompilerParams(
    use_tc_tiling_on_sc=False, needs_layout_passes=False)

_DN = (((0,), (1,)), ((), ()))


def _sc_mesh():
    return plsc.VectorSubcoreMesh(
        core_axis_name="c", subcore_axis_name="s",
        num_cores=_NC, num_subcores=_NS)


def _proj_body(x1t_ref, x2t_ref, wqt_ref, bq_ref, wkt_ref, bk_ref,
               wvt_ref, bv_ref, q_ref, kv_ref):
    q = lax.dot_general(x1t_ref[...], wqt_ref[...], _DN,
                        preferred_element_type=jnp.float32)
    q_ref[...] = q + bq_ref[...]
    kk = lax.dot_general(x2t_ref[...], wkt_ref[...], _DN,
                         preferred_element_type=jnp.float32)
    vv = lax.dot_general(x2t_ref[...], wvt_ref[...], _DN,
                         preferred_element_type=jnp.float32)
    kv_ref[:, : _QDO] = kk + bk_ref[...]
    kv_ref[:, _QDO:_KVD] = vv + bv_ref[...]


def _tc_proj(X1t, X2t, Wqt, bq2, Wkt, bk2, Wvt, bv2):
    g = 4
    return pl.pallas_call(
        _proj_body,
        grid=(g,),
        in_specs=[
            pl.BlockSpec((_QDI, _NQ // g), lambda i: (0, i)),
            pl.BlockSpec((_VDI, _NK // g), lambda i: (0, i)),
            pl.BlockSpec((_QDO, _QDI), lambda i: (0, 0)),
            pl.BlockSpec((1, _QDO), lambda i: (0, 0)),
            pl.BlockSpec((_QDO, _VDI), lambda i: (0, 0)),
            pl.BlockSpec((1, _QDO), lambda i: (0, 0)),
            pl.BlockSpec((_VDO, _VDI), lambda i: (0, 0)),
            pl.BlockSpec((1, _VDO), lambda i: (0, 0)),
        ],
        out_specs=[
            pl.BlockSpec((_NQ // g, _QDO), lambda i: (i, 0)),
            pl.BlockSpec((_NK // g, _W), lambda i: (i, 0)),
        ],
        out_shape=[
            jax.ShapeDtypeStruct((_NQ, _QDO), jnp.float32),
            jax.ShapeDtypeStruct((_NK, _W), jnp.float32),
        ],
    )(X1t, X2t, Wqt, bq2, Wkt, bk2, Wvt, bv2)


def _sc_attend(Q, KV, row_map):
    @functools.partial(
        pl.kernel,
        out_type=(
            jax.ShapeDtypeStruct((_NQ, _W), jnp.float32),
            jax.ShapeDtypeStruct((_NQ, _W), jnp.float32),
            jax.ShapeDtypeStruct((8, _W), jnp.float32),
        ),
        mesh=_sc_mesh(),
        scratch_types=[
            pltpu.VMEM((_NCH, _CH), jnp.int32),
            pltpu.VMEM((_KPW, _QDO), jnp.float32),
            pltpu.VMEM((_KPW, _W), jnp.float32),
            pltpu.VMEM((_KPW,), jnp.float32),
            pltpu.VMEM((_KPW, _CW), jnp.float32),
            pltpu.VMEM((64, _CW), jnp.float32),
            pltpu.VMEM((_L,), jnp.float32),
            pltpu.VMEM((_NS, _L), jnp.float32),
            pltpu.VMEM_SHARED((_NS, _L), jnp.float32),
            pltpu.VMEM_SHARED((_NQ, _CW), jnp.float32),
            pltpu.SemaphoreType.DMA,
            pltpu.SemaphoreType.DMA,
            pltpu.SemaphoreType.DMA,
            pltpu.SemaphoreType.DMA,
        ],
        compiler_params=_SC_PARAMS,
    )
    def k(q_hbm, kv_hbm, rm_hbm, out0_hbm, out1_hbm, mx_hbm,
          idx_v, qg_v, kv_v, s_v, ct_v, zb_v, mymax_v, allmax_v, mx_sh, acc_sh,
          sem_i, sem_k, sem_g, sem_z):
        c = lax.axis_index("c")
        s = lax.axis_index("s")
        w = s * _NC + c
        base = w * _KPW

        idx_cps = [
            pltpu.async_copy(rm_hbm.at[pl.ds(base + j * _CH, _CH)],
                             idx_v.at[j], sem_i)
            for j in range(_NCH)
        ]
        kv_cp = pltpu.async_copy(
            kv_hbm.at[pl.ds(base, _KPW)], kv_v, sem_k)

        g_cps = []
        for j in range(_NCH):
            idx_cps[j].wait()
            g_cps.append(pltpu.async_copy(
                q_hbm.at[idx_v.at[j]],
                qg_v.at[pl.ds(j * _CH, _CH)], sem_g))

        zero16 = jnp.zeros((_L,), jnp.float32)

        def zero_body(i, carry):
            zb_v[i, pl.ds(0, _L)] = zero16
            zb_v[i, pl.ds(_L, _L)] = zero16
            zb_v[i, pl.ds(2 * _L, _L)] = zero16
            return carry

        lax.fori_loop(0, 64, zero_body, 0)
        z_cps = [
            pltpu.async_copy(zb_v, acc_sh.at[pl.ds(s * _RPT + t * 64, 64)],
                             sem_z)
            for t in range(_RPT // 64)
        ]

        kv_cp.wait()
        for cp in g_cps:
            cp.wait()

        lane = lax.iota(jnp.int32, _L)
        last = lane == (_L - 1)

        def dot_body(j, carry):
            p = (qg_v[j, pl.ds(0, _L)] * kv_v[j, pl.ds(0, _L)]
                 + qg_v[j, pl.ds(_L, _L)] * kv_v[j, pl.ds(_L, _L)])
            tot = plsc.cumsum(p) * _ISQ
            plsc.store_scatter(s_v, [jnp.full((_L,), j, jnp.int32)], tot,
                               mask=last)
            return carry

        lax.fori_loop(0, _KPW, dot_body, 0, unroll=2)

        def max_body(g, mx):
            return jnp.maximum(mx, s_v[pl.ds(g * _L, _L)])

        mx = lax.fori_loop(1, _G, max_body, s_v[pl.ds(0, _L)])
        mymax_v[...] = jnp.broadcast_to(jnp.max(mx), (_L,))

        pltpu.sync_copy(mymax_v, mx_sh.at[s])
        plsc.subcore_barrier()
        pltpu.sync_copy(mx_sh, allmax_v)

        def cmax_body(t, mx):
            return jnp.maximum(mx, allmax_v[t, :])

        mcore = lax.fori_loop(1, _NS, cmax_body, allmax_v[0, :])

        @pl.when(s == 0)
        def _():
            mymax_v[...] = mcore
            pltpu.sync_copy(mymax_v, mx_hbm.at[c, pl.ds(0, _L)])

        def ct_body(g, carry):
            e16 = jnp.exp(s_v[pl.ds(g * _L, _L)] - mcore)
            plsc.store_scatter(s_v, [lane + g * _L], e16)
            return carry

        lax.fori_loop(0, _G, ct_body, 0)

        def row_body(j, carry):
            esp = plsc.load_gather(s_v, [jnp.full((_L,), j, jnp.int32)])
            ct_v[j, pl.ds(0, _L)] = esp * kv_v[j, pl.ds(_QDO, _L)]
            ct_v[j, pl.ds(_L, _L)] = esp * kv_v[j, pl.ds(_QDO + _L, _L)]
            ct_v[j, pl.ds(2 * _L, _L)] = esp
            return carry

        lax.fori_loop(0, _KPW, row_body, 0, unroll=2)

        for cp in z_cps:
            cp.wait()
        plsc.subcore_barrier()
        for j in range(_NCH):
            pltpu.sync_copy(ct_v.at[pl.ds(j * _CH, _CH)],
                            acc_sh.at[idx_v.at[j]], add=True)
        plsc.subcore_barrier()

        @pl.when(c == 0)
        def _():
            pltpu.sync_copy(acc_sh.at[pl.ds(s * _RPT, _RPT)],
                            out0_hbm.at[pl.ds(s * _RPT, _RPT), pl.ds(0, _CW)])

        @pl.when(c == 1)
        def _():
            pltpu.sync_copy(acc_sh.at[pl.ds(s * _RPT, _RPT)],
                            out1_hbm.at[pl.ds(s * _RPT, _RPT), pl.ds(0, _CW)])

    return k(Q, KV, row_map)


def _combine_body(p0_ref, p1_ref, mx_ref, c_ref):
    m0 = jnp.max(mx_ref[0:1, 0:_L])
    m1 = jnp.max(mx_ref[1:2, 0:_L])
    mg = jnp.maximum(m0, m1)
    a0 = jnp.exp(m0 - mg)
    a1 = jnp.exp(m1 - mg)
    num = a0 * p0_ref[:, :_VDO] + a1 * p1_ref[:, :_VDO]
    den = (a0 * p0_ref[:, _VDO:_VDO + 1] + a1 * p1_ref[:, _VDO:_VDO + 1])
    c_ref[...] = jnp.where(den != 0.0, num / den, 0.0).T


def _tc_combine(P0, P1, MX):
    return pl.pallas_call(
        _combine_body,
        in_specs=[
            pl.BlockSpec((_NQ, _W), lambda: (0, 0)),
            pl.BlockSpec((_NQ, _W), lambda: (0, 0)),
            pl.BlockSpec((8, _W), lambda: (0, 0)),
        ],
        out_specs=pl.BlockSpec((_VDO, _NQ), lambda: (0, 0)),
        out_shape=jax.ShapeDtypeStruct((_VDO, _NQ), jnp.float32),
    )(P0, P1, MX)


def kernel(X1, X2, row_map, Wq, bq, Wk, bk, Wv, bv):
    Q, KV = _tc_proj(X1.T, X2.T, Wq.T, bq.reshape(1, _QDO),
                     Wk.T, bk.reshape(1, _QDO), Wv.T, bv.reshape(1, _VDO))
    P0, P1, MX = _sc_attend(Q, KV, row_map)
    return _tc_combine(P0, P1, MX).T


# speedup vs baseline: 96.6415x; 1.2455x over previous
import functools

import jax
import jax.numpy as jnp
import numpy as np
from jax import lax
from jax.experimental import pallas as pl
from jax.experimental.pallas import tpu as pltpu
from jax.experimental.pallas import tpu_sc as plsc

_NQ = 4096
_NK = 16384
_QDI = 64
_QDO = 32
_VDI = 64
_VDO = 32
_KVD = _QDO + _VDO
_L = 16
_W = 128

_NC = 2
_NS = 16
_NW = _NC * _NS
_KPW = _NK // _NW
_CH = 128
_NCH = _KPW // _CH
_CW = _VDO + _L
_RPT = _NQ // _NS
_G = _KPW // _L

_ISQ = np.float32(1.0 / np.sqrt(np.float32(_QDO)))

_SC_PARAMS = pltpu.CompilerParams(
    use_tc_tiling_on_sc=False, needs_layout_passes=False)

_DN = (((0,), (1,)), ((), ()))


def _sc_mesh():
    return plsc.VectorSubcoreMesh(
        core_axis_name="c", subcore_axis_name="s",
        num_cores=_NC, num_subcores=_NS)


def _proj_body(x1t_ref, x2ta_ref, x2tb_ref, wqt_ref, bq_ref, wkt_ref, bk_ref,
               wvt_ref, bv_ref, q_ref, kv_ref):
    q = lax.dot_general(x1t_ref[...], wqt_ref[...], _DN,
                        preferred_element_type=jnp.float32)
    q_ref[...] = q + bq_ref[...]
    ka = lax.dot_general(x2ta_ref[...], wkt_ref[...], _DN,
                         preferred_element_type=jnp.float32)
    va = lax.dot_general(x2ta_ref[...], wvt_ref[...], _DN,
                         preferred_element_type=jnp.float32)
    kb = lax.dot_general(x2tb_ref[...], wkt_ref[...], _DN,
                         preferred_element_type=jnp.float32)
    vb = lax.dot_general(x2tb_ref[...], wvt_ref[...], _DN,
                         preferred_element_type=jnp.float32)
    kv_ref[:, 0:_QDO] = ka + bk_ref[...]
    kv_ref[:, _QDO:_KVD] = va + bv_ref[...]
    kv_ref[:, _KVD:_KVD + _QDO] = kb + bk_ref[...]
    kv_ref[:, _KVD + _QDO:_W] = vb + bv_ref[...]


def _tc_proj(X1t, X2t, Wqt, bq2, Wkt, bk2, Wvt, bv2):
    g = 4
    return pl.pallas_call(
        _proj_body,
        grid=(g,),
        in_specs=[
            pl.BlockSpec((_QDI, _NQ // g), lambda i: (0, i)),
            pl.BlockSpec((_VDI, _NK // (2 * g)), lambda i: (0, i)),
            pl.BlockSpec((_VDI, _NK // (2 * g)), lambda i: (0, i + g)),
            pl.BlockSpec((_QDO, _QDI), lambda i: (0, 0)),
            pl.BlockSpec((1, _QDO), lambda i: (0, 0)),
            pl.BlockSpec((_QDO, _VDI), lambda i: (0, 0)),
            pl.BlockSpec((1, _QDO), lambda i: (0, 0)),
            pl.BlockSpec((_VDO, _VDI), lambda i: (0, 0)),
            pl.BlockSpec((1, _VDO), lambda i: (0, 0)),
        ],
        out_specs=[
            pl.BlockSpec((_NQ // g, _QDO), lambda i: (i, 0)),
            pl.BlockSpec((_NK // (2 * g), _W), lambda i: (i, 0)),
        ],
        out_shape=[
            jax.ShapeDtypeStruct((_NQ, _QDO), jnp.float32),
            jax.ShapeDtypeStruct((_NK // 2, _W), jnp.float32),
        ],
    )(X1t, X2t, X2t, Wqt, bq2, Wkt, bk2, Wvt, bv2)


def _sc_attend(Q, KV, row_map):
    @functools.partial(
        pl.kernel,
        out_type=(
            jax.ShapeDtypeStruct((_NQ, _W), jnp.float32),
            jax.ShapeDtypeStruct((_NQ, _W), jnp.float32),
            jax.ShapeDtypeStruct((8, _W), jnp.float32),
        ),
        mesh=_sc_mesh(),
        scratch_types=[
            pltpu.VMEM((_NCH, _CH), jnp.int32),
            pltpu.VMEM((_KPW, _QDO), jnp.float32),
            pltpu.VMEM((_KPW, _W), jnp.float32),
            pltpu.VMEM((_KPW,), jnp.float32),
            pltpu.VMEM((_KPW,), jnp.float32),
            pltpu.VMEM((_KPW, _CW), jnp.float32),
            pltpu.VMEM((64, _CW), jnp.float32),
            pltpu.VMEM((_L,), jnp.float32),
            pltpu.VMEM((_NS, _L), jnp.float32),
            pltpu.VMEM_SHARED((_NS, _L), jnp.float32),
            pltpu.VMEM_SHARED((_NQ, _CW), jnp.float32),
            pltpu.SemaphoreType.DMA,
            pltpu.SemaphoreType.DMA,
            pltpu.SemaphoreType.DMA,
            pltpu.SemaphoreType.DMA,
        ],
        compiler_params=_SC_PARAMS,
    )
    def k(q_hbm, kv_hbm, rm_hbm, out0_hbm, out1_hbm, mx_hbm,
          idx_v, qg_v, kv_v, s_v, e_v, ct_v, zb_v, mymax_v, allmax_v, mx_sh, acc_sh,
          sem_i, sem_k, sem_g, sem_z):
        c = lax.axis_index("c")
        s = lax.axis_index("s")
        w = s * _NC + c
        base = w * _KPW

        idx_cps = [
            pltpu.async_copy(rm_hbm.at[pl.ds(base + j * _CH, _CH)],
                             idx_v.at[j], sem_i)
            for j in range(_NCH)
        ]
        half = base // (_NK // 2)
        coff = half * _KVD
        kv_cp = pltpu.async_copy(
            kv_hbm.at[pl.ds(base - half * (_NK // 2), _KPW)], kv_v, sem_k)

        g_cps = []
        for j in range(_NCH):
            idx_cps[j].wait()
            g_cps.append(pltpu.async_copy(
                q_hbm.at[idx_v.at[j]],
                qg_v.at[pl.ds(j * _CH, _CH)], sem_g))

        zero16 = jnp.zeros((_L,), jnp.float32)

        def zero_body(i, carry):
            zb_v[i, pl.ds(0, _L)] = zero16
            zb_v[i, pl.ds(_L, _L)] = zero16
            zb_v[i, pl.ds(2 * _L, _L)] = zero16
            return carry

        lax.fori_loop(0, 64, zero_body, 0)
        z_cps = [
            pltpu.async_copy(zb_v, acc_sh.at[pl.ds(s * _RPT + t * 64, 64)],
                             sem_z)
            for t in range(_RPT // 64)
        ]

        kv_cp.wait()
        for cp in g_cps:
            cp.wait()

        lane = lax.iota(jnp.int32, _L)
        last = lane == (_L - 1)

        @plsc.parallel_loop(0, _KPW, unroll=8)
        def dot_body(j):
            p = (qg_v[j, pl.ds(0, _L)] * kv_v[j, pl.ds(coff, _L)]
                 + qg_v[j, pl.ds(_L, _L)] * kv_v[j, pl.ds(coff + _L, _L)])
            tot = plsc.cumsum(p) * _ISQ
            plsc.store_scatter(s_v, [jnp.full((_L,), j, jnp.int32)], tot,
                               mask=last)

        def max_body(g, mx):
            return jnp.maximum(mx, s_v[pl.ds(g * _L, _L)])

        mx = lax.fori_loop(1, _G, max_body, s_v[pl.ds(0, _L)])
        mymax_v[...] = jnp.broadcast_to(jnp.max(mx), (_L,))

        pltpu.sync_copy(mymax_v, mx_sh.at[s])
        plsc.subcore_barrier()
        pltpu.sync_copy(mx_sh, allmax_v)

        def cmax_body(t, mx):
            return jnp.maximum(mx, allmax_v[t, :])

        mcore = lax.fori_loop(1, _NS, cmax_body, allmax_v[0, :])

        @pl.when(s == 0)
        def _():
            mymax_v[...] = mcore
            pltpu.sync_copy(mymax_v, mx_hbm.at[c, pl.ds(0, _L)])

        @plsc.parallel_loop(0, _G, unroll=2)
        def ct_body(g):
            e16 = jnp.exp(s_v[pl.ds(g * _L, _L)] - mcore)
            e_v[pl.ds(g * _L, _L)] = e16

        for cp in z_cps:
            cp.wait()
        plsc.subcore_barrier()

        sc_cps = []
        for ch in range(_NCH):
            @plsc.parallel_loop(ch * _CH, (ch + 1) * _CH, unroll=4)
            def row_body(j):
                esp = plsc.load_gather(e_v, [jnp.full((_L,), j, jnp.int32)])
                ct_v[j, pl.ds(0, _L)] = esp * kv_v[j, pl.ds(coff + _QDO, _L)]
                ct_v[j, pl.ds(_L, _L)] = esp * kv_v[j, pl.ds(coff + _QDO + _L, _L)]
                ct_v[j, pl.ds(2 * _L, _L)] = esp

            sc_cps.append(pltpu.async_copy(
                ct_v.at[pl.ds(ch * _CH, _CH)],
                acc_sh.at[idx_v.at[ch]], sem_z, add=True))
        for cp in sc_cps:
            cp.wait()
        plsc.subcore_barrier()

        @pl.when(c == 0)
        def _():
            pltpu.sync_copy(acc_sh.at[pl.ds(s * _RPT, _RPT)],
                            out0_hbm.at[pl.ds(s * _RPT, _RPT), pl.ds(0, _CW)])

        @pl.when(c == 1)
        def _():
            pltpu.sync_copy(acc_sh.at[pl.ds(s * _RPT, _RPT)],
                            out1_hbm.at[pl.ds(s * _RPT, _RPT), pl.ds(0, _CW)])

    return k(Q, KV, row_map)


def _combine_body(p0_ref, p1_ref, mx_ref, c_ref):
    m0 = jnp.max(mx_ref[0:1, 0:_L])
    m1 = jnp.max(mx_ref[1:2, 0:_L])
    mg = jnp.maximum(m0, m1)
    a0 = jnp.exp(m0 - mg)
    a1 = jnp.exp(m1 - mg)
    num = a0 * p0_ref[:, :_VDO] + a1 * p1_ref[:, :_VDO]
    den = (a0 * p0_ref[:, _VDO:_VDO + 1] + a1 * p1_ref[:, _VDO:_VDO + 1])
    c_ref[...] = jnp.where(den != 0.0, num / den, 0.0).T


def _tc_combine(P0, P1, MX):
    return pl.pallas_call(
        _combine_body,
        in_specs=[
            pl.BlockSpec((_NQ, _W), lambda: (0, 0)),
            pl.BlockSpec((_NQ, _W), lambda: (0, 0)),
            pl.BlockSpec((8, _W), lambda: (0, 0)),
        ],
        out_specs=pl.BlockSpec((_VDO, _NQ), lambda: (0, 0)),
        out_shape=jax.ShapeDtypeStruct((_VDO, _NQ), jnp.float32),
    )(P0, P1, MX)


def kernel(X1, X2, row_map, Wq, bq, Wk, bk, Wv, bv):
    Q, KV = _tc_proj(X1.T, X2.T, Wq.T, bq.reshape(1, _QDO),
                     Wk.T, bk.reshape(1, _QDO), Wv.T, bv.reshape(1, _VDO))
    P0, P1, MX = _sc_attend(Q, KV, row_map)
    return _tc_combine(P0, P1, MX).T
